# Initial kernel scaffold; baseline (speedup 1.0000x reference)
import jax
import jax.numpy as jnp
from jax.experimental import pallas as pl


def kernel(mat, seq, W_seq, gcn1_W, gcn1_b, lin1_W, ln1_g, ln1_b, gcn2_W, gcn2_b, lin2_W, ln2_g, ln2_b):
    raise NotImplementedError("write your pallas kernel here")


# speedup vs baseline: 7.8135x; 7.8135x over previous
import functools

import jax
import jax.numpy as jnp
from jax import lax
from jax.experimental import pallas as pl
from jax.experimental.pallas import tpu as pltpu
from jax.experimental.pallas import tpu_sc as plsc

N = 10000
E = 320000
D = 128

NC = 2
NS = 16
NW = NC * NS
CHUNK = 128
IB = 16
K = -(-E // (NW * CHUNK * IB)) * IB
NB = K // IB
E_PAD = NW * CHUNK * K
N_PAD = ((N + 1) + NS * CHUNK - 1) // (NS * CHUNK) * (NS * CHUNK)
RPT = N_PAD // NS

BR = 128
GRID = N_PAD // BR

_mesh = plsc.VectorSubcoreMesh(
    core_axis_name="c", subcore_axis_name="s", num_cores=NC, num_subcores=NS)


@functools.partial(
    pl.kernel,
    out_type=jax.ShapeDtypeStruct((NW, N_PAD), jnp.float32),
    mesh=_mesh,
    scratch_types=[
        pltpu.VMEM((IB, CHUNK), jnp.int32),
        pltpu.VMEM((N_PAD,), jnp.float32),
    ],
    compiler_params=pltpu.CompilerParams(needs_layout_passes=False),
)
def _deg_call(dst_hbm, out_hbm, idxb_v, hist_v):
    c = lax.axis_index("c")
    s = lax.axis_index("s")
    w = s * NC + c

    zeros16 = jnp.zeros((16,), jnp.float32)

    def zchunk(i, carry):
        hist_v[pl.ds(i * 16, 16)] = zeros16
        return carry

    lax.fori_loop(0, N_PAD // 16, zchunk, 0)

    ones16 = jnp.ones((16,), jnp.float32)

    def block(bi, carry):
        pltpu.sync_copy(dst_hbm.at[w, pl.ds(bi * IB, IB)], idxb_v)

        def chunk(j, inner):
            for q in range(CHUNK // 16):
                iv = idxb_v[j, pl.ds(q * 16, 16)]
                plsc.addupdate_scatter(hist_v, [iv], ones16)
            return inner

        lax.fori_loop(0, IB, chunk, 0)
        return carry

    lax.fori_loop(0, NB, block, 0)
    pltpu.sync_copy(hist_v, out_hbm.at[w])


@functools.partial(
    pl.kernel,
    out_type=jax.ShapeDtypeStruct((NC, N_PAD, D), jnp.float32),
    mesh=_mesh,
    scratch_types=[
        pltpu.VMEM((IB, CHUNK), jnp.int32),
        pltpu.VMEM((IB, CHUNK), jnp.int32),
        pltpu.VMEM((CHUNK, D), jnp.float32),
        pltpu.VMEM((CHUNK, D), jnp.float32),
        pltpu.VMEM_SHARED((N_PAD, D), jnp.float32),
        pltpu.SemaphoreType.DMA,
        pltpu.SemaphoreType.DMA,
    ],
)
def _agg_call(src_hbm, dst_hbm, y_hbm, zeros_hbm, out_hbm,
              src_v, dst_v, buf0, buf1, acc_s, sem0, sem1):
    c = lax.axis_index("c")
    s = lax.axis_index("s")
    w = s * NC + c
    r0 = s * RPT
    pltpu.sync_copy(zeros_hbm.at[pl.ds(r0, RPT)], acc_s.at[pl.ds(r0, RPT)])
    plsc.subcore_barrier()

    def block(bi, carry):
        pltpu.sync_copy(src_hbm.at[w, pl.ds(bi * IB, IB)], src_v)
        pltpu.sync_copy(dst_hbm.at[w, pl.ds(bi * IB, IB)], dst_v)

        def body(i, inner):
            j = i * 2
            cp0 = pltpu.async_copy(y_hbm.at[src_v.at[j]], buf0, sem0)
            cp1 = pltpu.async_copy(y_hbm.at[src_v.at[j + 1]], buf1, sem1)
            cp0.wait()
            pltpu.sync_copy(buf0, acc_s.at[dst_v.at[j]], add=True)
            cp1.wait()
            pltpu.sync_copy(buf1, acc_s.at[dst_v.at[j + 1]], add=True)
            return inner

        lax.fori_loop(0, IB // 2, body, 0)
        return carry

    lax.fori_loop(0, NB, block, 0)
    plsc.subcore_barrier()
    pltpu.sync_copy(acc_s.at[pl.ds(r0, RPT)], out_hbm.at[c, pl.ds(r0, RPT)])


def _diag_dis(degp):
    deg = jnp.sum(degp, axis=0, keepdims=True) + 1.0
    dis = lax.rsqrt(deg)
    r = lax.broadcasted_iota(jnp.int32, (BR, BR), 0)
    c = lax.broadcasted_iota(jnp.int32, (BR, BR), 1)
    return jnp.where(r == c, 1.0, 0.0) * dis


def _stage_a_body(seq_ref, wseq_ref, g1w_ref, l1w_ref, g1b_ref, degp_ref,
                  xseq_ref, y1_ref, xpre_ref):
    xs = jnp.maximum(
        jnp.dot(seq_ref[...], wseq_ref[...], preferred_element_type=jnp.float32),
        0.0)
    xseq_ref[...] = xs
    dd = _diag_dis(degp_ref[...])
    y1_ref[...] = jnp.dot(
        dd, jnp.dot(xs, g1w_ref[...], preferred_element_type=jnp.float32),
        preferred_element_type=jnp.float32)
    xpre_ref[...] = (jnp.dot(xs, l1w_ref[...],
                             preferred_element_type=jnp.float32)
                     + g1b_ref[...] + 1e-6)


def _layer_norm_block(x, g, b):
    mu = jnp.mean(x, axis=-1, keepdims=True)
    xc = x - mu
    var = jnp.mean(xc * xc, axis=-1, keepdims=True)
    return xc * lax.rsqrt(var + 1e-5) * g + b


def _stage_b_body(parts_ref, y1_ref, xpre_ref, degp_ref, g_ref, b_ref,
                  g2w_ref, l2w_ref, g2b_ref, y2_ref, xpre2_ref):
    dd = _diag_dis(degp_ref[...])
    acc = parts_ref[0] + parts_ref[1] + y1_ref[...]
    s = jnp.dot(dd, acc, preferred_element_type=jnp.float32) + xpre_ref[...]
    x = jnp.maximum(_layer_norm_block(s, g_ref[...], b_ref[...]), 0.0)
    y2_ref[...] = jnp.dot(
        dd, jnp.dot(x, g2w_ref[...], preferred_element_type=jnp.float32),
        preferred_element_type=jnp.float32)
    xpre2_ref[...] = (jnp.dot(x, l2w_ref[...],
                              preferred_element_type=jnp.float32)
                      + g2b_ref[...] + 1e-6)


def _stage_c_body(parts_ref, y2_ref, xpre2_ref, degp_ref, g_ref, b_ref,
                  out_ref):
    dd = _diag_dis(degp_ref[...])
    acc = parts_ref[0] + parts_ref[1] + y2_ref[...]
    s = jnp.dot(dd, acc, preferred_element_type=jnp.float32) + xpre2_ref[...]
    out_ref[...] = _layer_norm_block(s, g_ref[...], b_ref[...])


_row_spec = pl.BlockSpec((BR, D), lambda i: (i, 0))
_w_spec = pl.BlockSpec((D, D), lambda i: (0, 0))
_b_spec = pl.BlockSpec((1, D), lambda i: (0, 0))
_deg_spec = pl.BlockSpec((NW, BR), lambda i: (0, i))
_parts_spec = pl.BlockSpec((2, BR, D), lambda i: (0, i, 0))
_row_out = jax.ShapeDtypeStruct((N_PAD, D), jnp.float32)


def kernel(mat, seq, W_seq, gcn1_W, gcn1_b, lin1_W, ln1_g, ln1_b,
           gcn2_W, gcn2_b, lin2_W, ln2_g, ln2_b):
    src = mat[0]
    dst = mat[1]
    pad = E_PAD - E
    src_t = jnp.concatenate(
        [src, jnp.zeros((pad,), jnp.int32)]).reshape(NW, K, CHUNK)
    dst_t = jnp.concatenate(
        [dst, jnp.full((pad,), N, jnp.int32)]).reshape(NW, K, CHUNK)

    zeros_nd = jnp.zeros((N_PAD, D), jnp.float32)
    seq_p = jnp.concatenate(
        [seq, jnp.zeros((N_PAD - N, D), jnp.float32)], axis=0)

    deg_parts = _deg_call(dst_t)

    g1b = gcn1_b.reshape(1, D)
    g2b = gcn2_b.reshape(1, D)

    x_seq, y1, xpre1 = pl.pallas_call(
        _stage_a_body,
        grid=(GRID,),
        in_specs=[_row_spec, _w_spec, _w_spec, _w_spec, _b_spec, _deg_spec],
        out_specs=[_row_spec, _row_spec, _row_spec],
        out_shape=[_row_out, _row_out, _row_out],
    )(seq_p, W_seq, gcn1_W, lin1_W, g1b, deg_parts)

    parts1 = _agg_call(src_t, dst_t, y1, zeros_nd)

    y2, xpre2 = pl.pallas_call(
        _stage_b_body,
        grid=(GRID,),
        in_specs=[_parts_spec, _row_spec, _row_spec, _deg_spec,
                  _b_spec, _b_spec, _w_spec, _w_spec, _b_spec],
        out_specs=[_row_spec, _row_spec],
        out_shape=[_row_out, _row_out],
    )(parts1, y1, xpre1, deg_parts, ln1_g.reshape(1, D), ln1_b.reshape(1, D),
      gcn2_W, lin2_W, g2b)

    parts2 = _agg_call(src_t, dst_t, y2, zeros_nd)

    out = pl.pallas_call(
        _stage_c_body,
        grid=(GRID,),
        in_specs=[_parts_spec, _row_spec, _row_spec, _deg_spec,
                  _b_spec, _b_spec],
        out_specs=_row_spec,
        out_shape=_row_out,
    )(parts2, y2, xpre2, deg_parts, ln2_g.reshape(1, D), ln2_b.reshape(1, D))

    return (x_seq[:N], out[:N])


# speedup vs baseline: 7.9425x; 1.0165x over previous
import functools

import jax
import jax.numpy as jnp
from jax import lax
from jax.experimental import pallas as pl
from jax.experimental.pallas import tpu as pltpu
from jax.experimental.pallas import tpu_sc as plsc

N = 10000
E = 320000
D = 128

NC = 2
NS = 16
NW = NC * NS
CHUNK = 128
IB = 16
K = -(-E // (NW * CHUNK * IB)) * IB
NB = K // IB
E_PAD = NW * CHUNK * K
N_PAD = ((N + 1) + NS * CHUNK - 1) // (NS * CHUNK) * (NS * CHUNK)
RPT = N_PAD // NS

BR = 128
GRID = N_PAD // BR

_mesh = plsc.VectorSubcoreMesh(
    core_axis_name="c", subcore_axis_name="s", num_cores=NC, num_subcores=NS)


@functools.partial(
    pl.kernel,
    out_type=jax.ShapeDtypeStruct((NW, N_PAD), jnp.float32),
    mesh=_mesh,
    scratch_types=[
        pltpu.VMEM((IB, CHUNK), jnp.int32),
        pltpu.VMEM((N_PAD,), jnp.float32),
    ],
    compiler_params=pltpu.CompilerParams(needs_layout_passes=False),
)
def _deg_call(dst_hbm, out_hbm, idxb_v, hist_v):
    c = lax.axis_index("c")
    s = lax.axis_index("s")
    w = s * NC + c

    zeros16 = jnp.zeros((16,), jnp.float32)

    def zchunk(i, carry):
        hist_v[pl.ds(i * 16, 16)] = zeros16
        return carry

    lax.fori_loop(0, N_PAD // 16, zchunk, 0)

    ones16 = jnp.ones((16,), jnp.float32)

    def block(bi, carry):
        pltpu.sync_copy(dst_hbm.at[w, pl.ds(bi * IB, IB)], idxb_v)

        def chunk(j, inner):
            for q in range(CHUNK // 16):
                iv = idxb_v[j, pl.ds(q * 16, 16)]
                plsc.addupdate_scatter(hist_v, [iv], ones16)
            return inner

        lax.fori_loop(0, IB, chunk, 0)
        return carry

    lax.fori_loop(0, NB, block, 0)
    pltpu.sync_copy(hist_v, out_hbm.at[w])


@functools.partial(
    pl.kernel,
    out_type=jax.ShapeDtypeStruct((NC, N_PAD, D), jnp.float32),
    mesh=_mesh,
    scratch_types=[
        pltpu.VMEM((IB, CHUNK), jnp.int32),
        pltpu.VMEM((IB, CHUNK), jnp.int32),
        pltpu.VMEM((CHUNK, D), jnp.float32),
        pltpu.VMEM((CHUNK, D), jnp.float32),
        pltpu.VMEM_SHARED((N_PAD, D), jnp.float32),
        pltpu.SemaphoreType.DMA,
        pltpu.SemaphoreType.DMA,
    ],
)
def _agg_call(src_hbm, dst_hbm, y_hbm, zeros_hbm, out_hbm,
              src_v, dst_v, buf0, buf1, acc_s, sem0, sem1):
    c = lax.axis_index("c")
    s = lax.axis_index("s")
    w = s * NC + c
    r0 = s * RPT
    pltpu.sync_copy(zeros_hbm.at[pl.ds(r0, RPT)], acc_s.at[pl.ds(r0, RPT)])
    plsc.subcore_barrier()

    def block(bi, carry):
        pltpu.sync_copy(src_hbm.at[w, pl.ds(bi * IB, IB)], src_v)
        pltpu.sync_copy(dst_hbm.at[w, pl.ds(bi * IB, IB)], dst_v)

        def body(i, inner):
            j = i * 2
            cp0 = pltpu.async_copy(y_hbm.at[src_v.at[j]], buf0, sem0)
            cp1 = pltpu.async_copy(y_hbm.at[src_v.at[j + 1]], buf1, sem1)
            cp0.wait()
            pltpu.sync_copy(buf0, acc_s.at[dst_v.at[j]], add=True)
            cp1.wait()
            pltpu.sync_copy(buf1, acc_s.at[dst_v.at[j + 1]], add=True)
            return inner

        lax.fori_loop(0, IB // 2, body, 0)
        return carry

    lax.fori_loop(0, NB, block, 0)
    plsc.subcore_barrier()
    pltpu.sync_copy(acc_s.at[pl.ds(r0, RPT)], out_hbm.at[c, pl.ds(r0, RPT)])


def _col_dis(degp):
    deg = jnp.sum(degp, axis=0)[:, None] + 1.0
    return lax.rsqrt(deg)


def _stage_a_body(seq_ref, wseq_ref, g1w_ref, l1w_ref, g1b_ref, degp_ref,
                  xseq_ref, y1_ref, xpre_ref):
    xs = jnp.maximum(
        jnp.dot(seq_ref[...], wseq_ref[...], preferred_element_type=jnp.float32),
        0.0)
    xseq_ref[...] = xs
    dis = _col_dis(degp_ref[...])
    y1_ref[...] = dis * jnp.dot(xs, g1w_ref[...],
                                preferred_element_type=jnp.float32)
    xpre_ref[...] = (jnp.dot(xs, l1w_ref[...],
                             preferred_element_type=jnp.float32)
                     + g1b_ref[...] + 1e-6)


def _layer_norm_block(x, g, b):
    mu = jnp.mean(x, axis=-1, keepdims=True)
    xc = x - mu
    var = jnp.mean(xc * xc, axis=-1, keepdims=True)
    return xc * lax.rsqrt(var + 1e-5) * g + b


def _stage_b_body(parts_ref, y1_ref, xpre_ref, degp_ref, g_ref, b_ref,
                  g2w_ref, l2w_ref, g2b_ref, y2_ref, xpre2_ref):
    dis = _col_dis(degp_ref[...])
    acc = parts_ref[0] + parts_ref[1] + y1_ref[...]
    s = dis * acc + xpre_ref[...]
    x = jnp.maximum(_layer_norm_block(s, g_ref[...], b_ref[...]), 0.0)
    y2_ref[...] = dis * jnp.dot(x, g2w_ref[...],
                                preferred_element_type=jnp.float32)
    xpre2_ref[...] = (jnp.dot(x, l2w_ref[...],
                              preferred_element_type=jnp.float32)
                      + g2b_ref[...] + 1e-6)


def _stage_c_body(parts_ref, y2_ref, xpre2_ref, degp_ref, g_ref, b_ref,
                  out_ref):
    dis = _col_dis(degp_ref[...])
    acc = parts_ref[0] + parts_ref[1] + y2_ref[...]
    s = dis * acc + xpre2_ref[...]
    out_ref[...] = _layer_norm_block(s, g_ref[...], b_ref[...])


_row_spec = pl.BlockSpec((BR, D), lambda i: (i, 0))
_w_spec = pl.BlockSpec((D, D), lambda i: (0, 0))
_b_spec = pl.BlockSpec((1, D), lambda i: (0, 0))
_deg_spec = pl.BlockSpec((NW, BR), lambda i: (0, i))
_parts_spec = pl.BlockSpec((2, BR, D), lambda i: (0, i, 0))
_row_out = jax.ShapeDtypeStruct((N_PAD, D), jnp.float32)


def kernel(mat, seq, W_seq, gcn1_W, gcn1_b, lin1_W, ln1_g, ln1_b,
           gcn2_W, gcn2_b, lin2_W, ln2_g, ln2_b):
    src = mat[0]
    dst = mat[1]
    pad = E_PAD - E
    src_t = jnp.concatenate(
        [src, jnp.zeros((pad,), jnp.int32)]).reshape(NW, K, CHUNK)
    pad_dst = N + jnp.arange(pad, dtype=jnp.int32) % (N_PAD - N)
    dst_t = jnp.concatenate([dst, pad_dst]).reshape(NW, K, CHUNK)

    zeros_nd = jnp.zeros((N_PAD, D), jnp.float32)
    seq_p = jnp.concatenate(
        [seq, jnp.zeros((N_PAD - N, D), jnp.float32)], axis=0)

    deg_parts = _deg_call(dst_t)

    g1b = gcn1_b.reshape(1, D)
    g2b = gcn2_b.reshape(1, D)

    x_seq, y1, xpre1 = pl.pallas_call(
        _stage_a_body,
        grid=(GRID,),
        in_specs=[_row_spec, _w_spec, _w_spec, _w_spec, _b_spec, _deg_spec],
        out_specs=[_row_spec, _row_spec, _row_spec],
        out_shape=[_row_out, _row_out, _row_out],
    )(seq_p, W_seq, gcn1_W, lin1_W, g1b, deg_parts)

    parts1 = _agg_call(src_t, dst_t, y1, zeros_nd)

    y2, xpre2 = pl.pallas_call(
        _stage_b_body,
        grid=(GRID,),
        in_specs=[_parts_spec, _row_spec, _row_spec, _deg_spec,
                  _b_spec, _b_spec, _w_spec, _w_spec, _b_spec],
        out_specs=[_row_spec, _row_spec],
        out_shape=[_row_out, _row_out],
    )(parts1, y1, xpre1, deg_parts, ln1_g.reshape(1, D), ln1_b.reshape(1, D),
      gcn2_W, lin2_W, g2b)

    parts2 = _agg_call(src_t, dst_t, y2, zeros_nd)

    out = pl.pallas_call(
        _stage_c_body,
        grid=(GRID,),
        in_specs=[_parts_spec, _row_spec, _row_spec, _deg_spec,
                  _b_spec, _b_spec],
        out_specs=_row_spec,
        out_shape=_row_out,
    )(parts2, y2, xpre2, deg_parts, ln2_g.reshape(1, D), ln2_b.reshape(1, D))

    return (x_seq[:N], out[:N])


# speedup vs baseline: 18.9383x; 2.3844x over previous
import functools

import jax
import jax.numpy as jnp
from jax import lax
from jax.experimental import pallas as pl
from jax.experimental.pallas import tpu as pltpu
from jax.experimental.pallas import tpu_sc as plsc

N = 10000
E = 320000
D = 128

NC = 2
NS = 16
NW = NC * NS
CHUNK = 128
IB = 16
K = -(-E // (NW * CHUNK * IB)) * IB
NB = K // IB
E_PAD = NW * CHUNK * K
N_PAD = ((N + 1) + NS * CHUNK - 1) // (NS * CHUNK) * (NS * CHUNK)
RPT = N_PAD // NS

BR = 128
GRID = N_PAD // BR

_mesh = plsc.VectorSubcoreMesh(
    core_axis_name="c", subcore_axis_name="s", num_cores=NC, num_subcores=NS)


@functools.partial(
    pl.kernel,
    out_type=jax.ShapeDtypeStruct((NW, N_PAD), jnp.float32),
    mesh=_mesh,
    scratch_types=[
        pltpu.VMEM((IB, CHUNK), jnp.int32),
        pltpu.VMEM((N_PAD,), jnp.float32),
    ],
    compiler_params=pltpu.CompilerParams(needs_layout_passes=False),
)
def _deg_call(dst_hbm, out_hbm, idxb_v, hist_v):
    c = lax.axis_index("c")
    s = lax.axis_index("s")
    w = s * NC + c

    zeros16 = jnp.zeros((16,), jnp.float32)

    def zchunk(i, carry):
        hist_v[pl.ds(i * 16, 16)] = zeros16
        return carry

    lax.fori_loop(0, N_PAD // 16, zchunk, 0)

    ones16 = jnp.ones((16,), jnp.float32)

    def block(bi, carry):
        pltpu.sync_copy(dst_hbm.at[w, pl.ds(bi * IB, IB)], idxb_v)

        def chunk(j, inner):
            for q in range(CHUNK // 16):
                iv = idxb_v[j, pl.ds(q * 16, 16)]
                plsc.addupdate_scatter(hist_v, [iv], ones16)
            return inner

        lax.fori_loop(0, IB, chunk, 0)
        return carry

    lax.fori_loop(0, NB, block, 0)
    pltpu.sync_copy(hist_v, out_hbm.at[w])


@functools.partial(
    pl.kernel,
    out_type=jax.ShapeDtypeStruct((NC, N_PAD, D), jnp.float32),
    mesh=_mesh,
    scratch_types=[
        pltpu.VMEM((IB, CHUNK), jnp.int32),
        pltpu.VMEM((IB, CHUNK), jnp.int32),
        pltpu.VMEM((CHUNK, D), jnp.float32),
        pltpu.VMEM((CHUNK, D), jnp.float32),
        pltpu.VMEM_SHARED((N_PAD, D), jnp.float32),
        pltpu.SemaphoreType.DMA,
        pltpu.SemaphoreType.DMA,
    ],
)
def _agg_call(src_hbm, dst_hbm, y_hbm, zeros_hbm, out_hbm,
              src_v, dst_v, buf0, buf1, acc_s, sem0, sem1):
    c = lax.axis_index("c")
    s = lax.axis_index("s")
    w = s * NC + c
    r0 = s * RPT
    pltpu.sync_copy(zeros_hbm.at[pl.ds(r0, RPT)], acc_s.at[pl.ds(r0, RPT)])
    plsc.subcore_barrier()

    def block(bi, carry):
        pltpu.sync_copy(src_hbm.at[w, pl.ds(bi * IB, IB)], src_v)
        pltpu.sync_copy(dst_hbm.at[w, pl.ds(bi * IB, IB)], dst_v)

        def body(i, inner):
            j = i * 2
            cp0 = pltpu.async_copy(y_hbm.at[src_v.at[j]], buf0, sem0)
            cp1 = pltpu.async_copy(y_hbm.at[src_v.at[j + 1]], buf1, sem1)
            cp0.wait()
            pltpu.sync_copy(buf0, acc_s.at[dst_v.at[j]], add=True)
            cp1.wait()
            pltpu.sync_copy(buf1, acc_s.at[dst_v.at[j + 1]], add=True)
            return inner

        lax.fori_loop(0, IB // 2, body, 0)
        return carry

    lax.fori_loop(0, NB, block, 0)
    plsc.subcore_barrier()
    pltpu.sync_copy(acc_s.at[pl.ds(r0, RPT)], out_hbm.at[c, pl.ds(r0, RPT)])


def _col_dis(degp):
    deg = jnp.sum(degp, axis=0)[:, None] + 1.0
    return lax.rsqrt(deg)


def _stage_a_body(seq_ref, wseq_ref, g1w_ref, l1w_ref, g1b_ref, degp_ref,
                  xseq_ref, y1_ref, xpre_ref):
    xs = jnp.maximum(
        jnp.dot(seq_ref[...], wseq_ref[...], preferred_element_type=jnp.float32),
        0.0)
    xseq_ref[...] = xs
    dis = _col_dis(degp_ref[...])
    y1_ref[...] = dis * jnp.dot(xs, g1w_ref[...],
                                preferred_element_type=jnp.float32)
    xpre_ref[...] = (jnp.dot(xs, l1w_ref[...],
                             preferred_element_type=jnp.float32)
                     + g1b_ref[...] + 1e-6)


def _layer_norm_block(x, g, b):
    mu = jnp.mean(x, axis=-1, keepdims=True)
    xc = x - mu
    var = jnp.mean(xc * xc, axis=-1, keepdims=True)
    return xc * lax.rsqrt(var + 1e-5) * g + b


def _stage_b_body(parts_ref, y1_ref, xpre_ref, degp_ref, g_ref, b_ref,
                  g2w_ref, l2w_ref, g2b_ref, y2_ref, xpre2_ref):
    dis = _col_dis(degp_ref[...])
    acc = parts_ref[0] + parts_ref[1] + y1_ref[...]
    s = dis * acc + xpre_ref[...]
    x = jnp.maximum(_layer_norm_block(s, g_ref[...], b_ref[...]), 0.0)
    y2_ref[...] = dis * jnp.dot(x, g2w_ref[...],
                                preferred_element_type=jnp.float32)
    xpre2_ref[...] = (jnp.dot(x, l2w_ref[...],
                              preferred_element_type=jnp.float32)
                      + g2b_ref[...] + 1e-6)


def _stage_c_body(parts_ref, y2_ref, xpre2_ref, degp_ref, g_ref, b_ref,
                  out_ref):
    dis = _col_dis(degp_ref[...])
    acc = parts_ref[0] + parts_ref[1] + y2_ref[...]
    s = dis * acc + xpre2_ref[...]
    out_ref[...] = _layer_norm_block(s, g_ref[...], b_ref[...])


_row_spec = pl.BlockSpec((BR, D), lambda i: (i, 0))
_w_spec = pl.BlockSpec((D, D), lambda i: (0, 0))
_b_spec = pl.BlockSpec((1, D), lambda i: (0, 0))
_deg_spec = pl.BlockSpec((NW, BR), lambda i: (0, i))
_parts_spec = pl.BlockSpec((2, BR, D), lambda i: (0, i, 0))
_row_out = jax.ShapeDtypeStruct((N_PAD, D), jnp.float32)


def kernel(mat, seq, W_seq, gcn1_W, gcn1_b, lin1_W, ln1_g, ln1_b,
           gcn2_W, gcn2_b, lin2_W, ln2_g, ln2_b):
    src = mat[0]
    dst = mat[1]
    pad = E_PAD - E
    pad_idx = N + jnp.arange(pad, dtype=jnp.int32) % (N_PAD - N)
    src_t = jnp.concatenate([src, pad_idx]).reshape(
        K * CHUNK, NW).T.reshape(NW, K, CHUNK)
    dst_t = jnp.concatenate([dst, pad_idx]).reshape(
        K * CHUNK, NW).T.reshape(NW, K, CHUNK)

    zeros_nd = jnp.zeros((N_PAD, D), jnp.float32)
    seq_p = jnp.concatenate(
        [seq, jnp.zeros((N_PAD - N, D), jnp.float32)], axis=0)

    deg_parts = _deg_call(dst_t)

    g1b = gcn1_b.reshape(1, D)
    g2b = gcn2_b.reshape(1, D)

    x_seq, y1, xpre1 = pl.pallas_call(
        _stage_a_body,
        grid=(GRID,),
        in_specs=[_row_spec, _w_spec, _w_spec, _w_spec, _b_spec, _deg_spec],
        out_specs=[_row_spec, _row_spec, _row_spec],
        out_shape=[_row_out, _row_out, _row_out],
    )(seq_p, W_seq, gcn1_W, lin1_W, g1b, deg_parts)

    parts1 = _agg_call(src_t, dst_t, y1, zeros_nd)

    y2, xpre2 = pl.pallas_call(
        _stage_b_body,
        grid=(GRID,),
        in_specs=[_parts_spec, _row_spec, _row_spec, _deg_spec,
                  _b_spec, _b_spec, _w_spec, _w_spec, _b_spec],
        out_specs=[_row_spec, _row_spec],
        out_shape=[_row_out, _row_out],
    )(parts1, y1, xpre1, deg_parts, ln1_g.reshape(1, D), ln1_b.reshape(1, D),
      gcn2_W, lin2_W, g2b)

    parts2 = _agg_call(src_t, dst_t, y2, zeros_nd)

    out = pl.pallas_call(
        _stage_c_body,
        grid=(GRID,),
        in_specs=[_parts_spec, _row_spec, _row_spec, _deg_spec,
                  _b_spec, _b_spec],
        out_specs=_row_spec,
        out_shape=_row_out,
    )(parts2, y2, xpre2, deg_parts, ln2_g.reshape(1, D), ln2_b.reshape(1, D))

    return (x_seq[:N], out[:N])


# speedup vs baseline: 19.2539x; 1.0167x over previous
import functools

import jax
import jax.numpy as jnp
from jax import lax
from jax.experimental import pallas as pl
from jax.experimental.pallas import tpu as pltpu
from jax.experimental.pallas import tpu_sc as plsc

N = 10000
E = 320000
D = 128

NC = 2
NS = 16
NW = NC * NS
CHUNK = 128
IB = 16
K = -(-E // (NW * CHUNK * IB)) * IB
NB = K // IB
E_PAD = NW * CHUNK * K
N_PAD = ((N + 1) + NS * CHUNK - 1) // (NS * CHUNK) * (NS * CHUNK)
RPT = N_PAD // NS

BR = 128
GRID = N_PAD // BR

_mesh = plsc.VectorSubcoreMesh(
    core_axis_name="c", subcore_axis_name="s", num_cores=NC, num_subcores=NS)


@functools.partial(
    pl.kernel,
    out_type=jax.ShapeDtypeStruct((NW, N_PAD), jnp.float32),
    mesh=_mesh,
    scratch_types=[
        pltpu.VMEM((IB, CHUNK), jnp.int32),
        pltpu.VMEM((N_PAD,), jnp.float32),
    ],
    compiler_params=pltpu.CompilerParams(needs_layout_passes=False),
)
def _deg_call(dst_hbm, out_hbm, idxb_v, hist_v):
    c = lax.axis_index("c")
    s = lax.axis_index("s")
    w = s * NC + c

    zeros16 = jnp.zeros((16,), jnp.float32)

    def zchunk(i, carry):
        hist_v[pl.ds(i * 16, 16)] = zeros16
        return carry

    lax.fori_loop(0, N_PAD // 16, zchunk, 0)

    ones16 = jnp.ones((16,), jnp.float32)

    def block(bi, carry):
        pltpu.sync_copy(dst_hbm.at[w, pl.ds(bi * IB, IB)], idxb_v)

        def chunk(j, inner):
            for q in range(CHUNK // 16):
                iv = idxb_v[j, pl.ds(q * 16, 16)]
                plsc.addupdate_scatter(hist_v, [iv], ones16)
            return inner

        lax.fori_loop(0, IB, chunk, 0)
        return carry

    lax.fori_loop(0, NB, block, 0)
    pltpu.sync_copy(hist_v, out_hbm.at[w])


@functools.partial(
    pl.kernel,
    out_type=jax.ShapeDtypeStruct((NC, N_PAD, D), jnp.float32),
    mesh=_mesh,
    scratch_types=[
        pltpu.VMEM((IB, CHUNK), jnp.int32),
        pltpu.VMEM((IB, CHUNK), jnp.int32),
        pltpu.VMEM((CHUNK, D), jnp.float32),
        pltpu.VMEM((CHUNK, D), jnp.float32),
        pltpu.VMEM_SHARED((N_PAD, D), jnp.float32),
        pltpu.SemaphoreType.DMA,
        pltpu.SemaphoreType.DMA,
        pltpu.SemaphoreType.DMA,
        pltpu.SemaphoreType.DMA,
    ],
)
def _agg_call(src_hbm, dst_hbm, y_hbm, zeros_hbm, out_hbm,
              src_v, dst_v, buf0, buf1, acc_s, gsem0, gsem1, ssem0, ssem1):
    c = lax.axis_index("c")
    s = lax.axis_index("s")
    w = s * NC + c
    r0 = s * RPT
    pltpu.sync_copy(zeros_hbm.at[pl.ds(r0, RPT)], acc_s.at[pl.ds(r0, RPT)])
    plsc.subcore_barrier()

    def block(bi, carry):
        pltpu.sync_copy(src_hbm.at[w, pl.ds(bi * IB, IB)], src_v)
        pltpu.sync_copy(dst_hbm.at[w, pl.ds(bi * IB, IB)], dst_v)

        pltpu.async_copy(y_hbm.at[src_v.at[0]], buf0, gsem0)
        pltpu.async_copy(y_hbm.at[src_v.at[1]], buf1, gsem1)

        def body(i, inner):
            j = i * 2
            pltpu.make_async_copy(y_hbm.at[src_v.at[j]], buf0, gsem0).wait()
            sc0 = pltpu.async_copy(
                buf0, acc_s.at[dst_v.at[j]], ssem0, add=True)
            pltpu.make_async_copy(y_hbm.at[src_v.at[j + 1]], buf1,
                                  gsem1).wait()
            sc1 = pltpu.async_copy(
                buf1, acc_s.at[dst_v.at[j + 1]], ssem1, add=True)
            sc0.wait()
            pltpu.async_copy(y_hbm.at[src_v.at[j + 2]], buf0, gsem0)
            sc1.wait()
            pltpu.async_copy(y_hbm.at[src_v.at[j + 3]], buf1, gsem1)
            return inner

        lax.fori_loop(0, IB // 2 - 1, body, 0)

        j = IB - 2
        pltpu.make_async_copy(y_hbm.at[src_v.at[j]], buf0, gsem0).wait()
        sc0 = pltpu.async_copy(buf0, acc_s.at[dst_v.at[j]], ssem0, add=True)
        pltpu.make_async_copy(y_hbm.at[src_v.at[j + 1]], buf1, gsem1).wait()
        sc1 = pltpu.async_copy(
            buf1, acc_s.at[dst_v.at[j + 1]], ssem1, add=True)
        sc0.wait()
        sc1.wait()
        return carry

    lax.fori_loop(0, NB, block, 0)
    plsc.subcore_barrier()
    pltpu.sync_copy(acc_s.at[pl.ds(r0, RPT)], out_hbm.at[c, pl.ds(r0, RPT)])


def _col_dis(degp):
    deg = jnp.sum(degp, axis=0)[:, None] + 1.0
    return lax.rsqrt(deg)


def _stage_a_body(seq_ref, wseq_ref, g1w_ref, l1w_ref, g1b_ref, degp_ref,
                  xseq_ref, y1_ref, xpre_ref):
    xs = jnp.maximum(
        jnp.dot(seq_ref[...], wseq_ref[...], preferred_element_type=jnp.float32),
        0.0)
    xseq_ref[...] = xs
    dis = _col_dis(degp_ref[...])
    y1_ref[...] = dis * jnp.dot(xs, g1w_ref[...],
                                preferred_element_type=jnp.float32)
    xpre_ref[...] = (jnp.dot(xs, l1w_ref[...],
                             preferred_element_type=jnp.float32)
                     + g1b_ref[...] + 1e-6)


def _layer_norm_block(x, g, b):
    mu = jnp.mean(x, axis=-1, keepdims=True)
    xc = x - mu
    var = jnp.mean(xc * xc, axis=-1, keepdims=True)
    return xc * lax.rsqrt(var + 1e-5) * g + b


def _stage_b_body(parts_ref, y1_ref, xpre_ref, degp_ref, g_ref, b_ref,
                  g2w_ref, l2w_ref, g2b_ref, y2_ref, xpre2_ref):
    dis = _col_dis(degp_ref[...])
    acc = parts_ref[0] + parts_ref[1] + y1_ref[...]
    s = dis * acc + xpre_ref[...]
    x = jnp.maximum(_layer_norm_block(s, g_ref[...], b_ref[...]), 0.0)
    y2_ref[...] = dis * jnp.dot(x, g2w_ref[...],
                                preferred_element_type=jnp.float32)
    xpre2_ref[...] = (jnp.dot(x, l2w_ref[...],
                              preferred_element_type=jnp.float32)
                      + g2b_ref[...] + 1e-6)


def _stage_c_body(parts_ref, y2_ref, xpre2_ref, degp_ref, g_ref, b_ref,
                  out_ref):
    dis = _col_dis(degp_ref[...])
    acc = parts_ref[0] + parts_ref[1] + y2_ref[...]
    s = dis * acc + xpre2_ref[...]
    out_ref[...] = _layer_norm_block(s, g_ref[...], b_ref[...])


_row_spec = pl.BlockSpec((BR, D), lambda i: (i, 0))
_w_spec = pl.BlockSpec((D, D), lambda i: (0, 0))
_b_spec = pl.BlockSpec((1, D), lambda i: (0, 0))
_deg_spec = pl.BlockSpec((NW, BR), lambda i: (0, i))
_parts_spec = pl.BlockSpec((2, BR, D), lambda i: (0, i, 0))
_row_out = jax.ShapeDtypeStruct((N_PAD, D), jnp.float32)


def kernel(mat, seq, W_seq, gcn1_W, gcn1_b, lin1_W, ln1_g, ln1_b,
           gcn2_W, gcn2_b, lin2_W, ln2_g, ln2_b):
    src = mat[0]
    dst = mat[1]
    pad = E_PAD - E
    pad_idx = N + jnp.arange(pad, dtype=jnp.int32) % (N_PAD - N)
    src_t = jnp.concatenate([src, pad_idx]).reshape(
        K * CHUNK, NW).T.reshape(NW, K, CHUNK)
    dst_t = jnp.concatenate([dst, pad_idx]).reshape(
        K * CHUNK, NW).T.reshape(NW, K, CHUNK)

    zeros_nd = jnp.zeros((N_PAD, D), jnp.float32)
    seq_p = jnp.concatenate(
        [seq, jnp.zeros((N_PAD - N, D), jnp.float32)], axis=0)

    deg_parts = _deg_call(dst_t)

    g1b = gcn1_b.reshape(1, D)
    g2b = gcn2_b.reshape(1, D)

    x_seq, y1, xpre1 = pl.pallas_call(
        _stage_a_body,
        grid=(GRID,),
        in_specs=[_row_spec, _w_spec, _w_spec, _w_spec, _b_spec, _deg_spec],
        out_specs=[_row_spec, _row_spec, _row_spec],
        out_shape=[_row_out, _row_out, _row_out],
    )(seq_p, W_seq, gcn1_W, lin1_W, g1b, deg_parts)

    parts1 = _agg_call(src_t, dst_t, y1, zeros_nd)

    y2, xpre2 = pl.pallas_call(
        _stage_b_body,
        grid=(GRID,),
        in_specs=[_parts_spec, _row_spec, _row_spec, _deg_spec,
                  _b_spec, _b_spec, _w_spec, _w_spec, _b_spec],
        out_specs=[_row_spec, _row_spec],
        out_shape=[_row_out, _row_out],
    )(parts1, y1, xpre1, deg_parts, ln1_g.reshape(1, D), ln1_b.reshape(1, D),
      gcn2_W, lin2_W, g2b)

    parts2 = _agg_call(src_t, dst_t, y2, zeros_nd)

    out = pl.pallas_call(
        _stage_c_body,
        grid=(GRID,),
        in_specs=[_parts_spec, _row_spec, _row_spec, _deg_spec,
                  _b_spec, _b_spec],
        out_specs=_row_spec,
        out_shape=_row_out,
    )(parts2, y2, xpre2, deg_parts, ln2_g.reshape(1, D), ln2_b.reshape(1, D))

    return (x_seq[:N], out[:N])


# speedup vs baseline: 23.8630x; 1.2394x over previous
import functools

import jax
import jax.numpy as jnp
from jax import lax
from jax.experimental import pallas as pl
from jax.experimental.pallas import tpu as pltpu
from jax.experimental.pallas import tpu_sc as plsc

N = 10000
E = 320000
D = 128

NC = 2
NS = 16
NW = NC * NS
CHUNK = 128
IB = 16
K = -(-E // (NW * CHUNK * IB)) * IB
NB = K // IB
E_PAD = NW * CHUNK * K
N_PAD = ((N + 1) + NS * CHUNK - 1) // (NS * CHUNK) * (NS * CHUNK)
RPT = N_PAD // NS

BR = 512
GRID = N_PAD // BR

_mesh = plsc.VectorSubcoreMesh(
    core_axis_name="c", subcore_axis_name="s", num_cores=NC, num_subcores=NS)


@functools.partial(
    pl.kernel,
    out_type=jax.ShapeDtypeStruct((NW, N_PAD), jnp.float32),
    mesh=_mesh,
    scratch_types=[
        pltpu.VMEM((IB, CHUNK), jnp.int32),
        pltpu.VMEM((N_PAD,), jnp.float32),
    ],
    compiler_params=pltpu.CompilerParams(needs_layout_passes=False),
)
def _deg_call(dst_hbm, out_hbm, idxb_v, hist_v):
    c = lax.axis_index("c")
    s = lax.axis_index("s")
    w = s * NC + c

    zeros16 = jnp.zeros((16,), jnp.float32)

    def zchunk(i, carry):
        hist_v[pl.ds(i * 16, 16)] = zeros16
        return carry

    lax.fori_loop(0, N_PAD // 16, zchunk, 0)

    ones16 = jnp.ones((16,), jnp.float32)

    def block(bi, carry):
        pltpu.sync_copy(dst_hbm.at[w, pl.ds(bi * IB, IB)], idxb_v)

        def chunk(j, inner):
            for q in range(CHUNK // 16):
                iv = idxb_v[j, pl.ds(q * 16, 16)]
                plsc.addupdate_scatter(hist_v, [iv], ones16)
            return inner

        lax.fori_loop(0, IB, chunk, 0)
        return carry

    lax.fori_loop(0, NB, block, 0)
    pltpu.sync_copy(hist_v, out_hbm.at[w])


@functools.partial(
    pl.kernel,
    out_type=jax.ShapeDtypeStruct((NC, N_PAD, D), jnp.float32),
    mesh=_mesh,
    scratch_types=[
        pltpu.VMEM((IB, CHUNK), jnp.int32),
        pltpu.VMEM((IB, CHUNK), jnp.int32),
        pltpu.VMEM((CHUNK, D), jnp.float32),
        pltpu.VMEM((CHUNK, D), jnp.float32),
        pltpu.VMEM_SHARED((N_PAD, D), jnp.float32),
        pltpu.SemaphoreType.DMA,
        pltpu.SemaphoreType.DMA,
        pltpu.SemaphoreType.DMA,
        pltpu.SemaphoreType.DMA,
    ],
)
def _agg_call(src_hbm, dst_hbm, y_hbm, zeros_hbm, out_hbm,
              src_v, dst_v, buf0, buf1, acc_s, gsem0, gsem1, ssem0, ssem1):
    c = lax.axis_index("c")
    s = lax.axis_index("s")
    w = s * NC + c
    r0 = s * RPT
    pltpu.sync_copy(zeros_hbm.at[pl.ds(r0, RPT)], acc_s.at[pl.ds(r0, RPT)])
    plsc.subcore_barrier()

    def block(bi, carry):
        pltpu.sync_copy(src_hbm.at[w, pl.ds(bi * IB, IB)], src_v)
        pltpu.sync_copy(dst_hbm.at[w, pl.ds(bi * IB, IB)], dst_v)

        pltpu.async_copy(y_hbm.at[src_v.at[0]], buf0, gsem0)
        pltpu.async_copy(y_hbm.at[src_v.at[1]], buf1, gsem1)

        def body(i, inner):
            j = i * 2
            pltpu.make_async_copy(y_hbm.at[src_v.at[j]], buf0, gsem0).wait()
            sc0 = pltpu.async_copy(
                buf0, acc_s.at[dst_v.at[j]], ssem0, add=True)
            pltpu.make_async_copy(y_hbm.at[src_v.at[j + 1]], buf1,
                                  gsem1).wait()
            sc1 = pltpu.async_copy(
                buf1, acc_s.at[dst_v.at[j + 1]], ssem1, add=True)
            sc0.wait()
            pltpu.async_copy(y_hbm.at[src_v.at[j + 2]], buf0, gsem0)
            sc1.wait()
            pltpu.async_copy(y_hbm.at[src_v.at[j + 3]], buf1, gsem1)
            return inner

        lax.fori_loop(0, IB // 2 - 1, body, 0)

        j = IB - 2
        pltpu.make_async_copy(y_hbm.at[src_v.at[j]], buf0, gsem0).wait()
        sc0 = pltpu.async_copy(buf0, acc_s.at[dst_v.at[j]], ssem0, add=True)
        pltpu.make_async_copy(y_hbm.at[src_v.at[j + 1]], buf1, gsem1).wait()
        sc1 = pltpu.async_copy(
            buf1, acc_s.at[dst_v.at[j + 1]], ssem1, add=True)
        sc0.wait()
        sc1.wait()
        return carry

    lax.fori_loop(0, NB, block, 0)
    plsc.subcore_barrier()
    pltpu.sync_copy(acc_s.at[pl.ds(r0, RPT)], out_hbm.at[c, pl.ds(r0, RPT)])


def _col_dis(degp):
    deg = jnp.sum(degp, axis=0)[:, None] + 1.0
    return lax.rsqrt(deg)


def _stage_a_body(seq_ref, wseq_ref, g1w_ref, l1w_ref, g1b_ref, degp_ref,
                  xseq_ref, y1_ref, xpre_ref):
    xs = jnp.maximum(
        jnp.dot(seq_ref[...], wseq_ref[...], preferred_element_type=jnp.float32),
        0.0)
    xseq_ref[...] = xs
    dis = _col_dis(degp_ref[...])
    y1_ref[...] = dis * jnp.dot(xs, g1w_ref[...],
                                preferred_element_type=jnp.float32)
    xpre_ref[...] = (jnp.dot(xs, l1w_ref[...],
                             preferred_element_type=jnp.float32)
                     + g1b_ref[...] + 1e-6)


def _layer_norm_block(x, g, b):
    mu = jnp.mean(x, axis=-1, keepdims=True)
    xc = x - mu
    var = jnp.mean(xc * xc, axis=-1, keepdims=True)
    return xc * lax.rsqrt(var + 1e-5) * g + b


def _stage_b_body(parts_ref, y1_ref, xpre_ref, degp_ref, g_ref, b_ref,
                  g2w_ref, l2w_ref, g2b_ref, y2_ref, xpre2_ref):
    dis = _col_dis(degp_ref[...])
    acc = parts_ref[0] + parts_ref[1] + y1_ref[...]
    s = dis * acc + xpre_ref[...]
    x = jnp.maximum(_layer_norm_block(s, g_ref[...], b_ref[...]), 0.0)
    y2_ref[...] = dis * jnp.dot(x, g2w_ref[...],
                                preferred_element_type=jnp.float32)
    xpre2_ref[...] = (jnp.dot(x, l2w_ref[...],
                              preferred_element_type=jnp.float32)
                      + g2b_ref[...] + 1e-6)


def _stage_c_body(parts_ref, y2_ref, xpre2_ref, degp_ref, g_ref, b_ref,
                  out_ref):
    dis = _col_dis(degp_ref[...])
    acc = parts_ref[0] + parts_ref[1] + y2_ref[...]
    s = dis * acc + xpre2_ref[...]
    out_ref[...] = _layer_norm_block(s, g_ref[...], b_ref[...])


_row_spec = pl.BlockSpec((BR, D), lambda i: (i, 0))
_w_spec = pl.BlockSpec((D, D), lambda i: (0, 0))
_b_spec = pl.BlockSpec((1, D), lambda i: (0, 0))
_deg_spec = pl.BlockSpec((NW, BR), lambda i: (0, i))
_parts_spec = pl.BlockSpec((2, BR, D), lambda i: (0, i, 0))
_row_out = jax.ShapeDtypeStruct((N_PAD, D), jnp.float32)


def kernel(mat, seq, W_seq, gcn1_W, gcn1_b, lin1_W, ln1_g, ln1_b,
           gcn2_W, gcn2_b, lin2_W, ln2_g, ln2_b):
    src = mat[0]
    dst = mat[1]
    pad = E_PAD - E
    pad_idx = N + jnp.arange(pad, dtype=jnp.int32) % (N_PAD - N)
    src_t = jnp.concatenate([src, pad_idx]).reshape(
        K * CHUNK, NW).T.reshape(NW, K, CHUNK)
    dst_t = jnp.concatenate([dst, pad_idx]).reshape(
        K * CHUNK, NW).T.reshape(NW, K, CHUNK)

    zeros_nd = jnp.zeros((N_PAD, D), jnp.float32)
    seq_p = jnp.concatenate(
        [seq, jnp.zeros((N_PAD - N, D), jnp.float32)], axis=0)

    deg_parts = _deg_call(dst_t)

    g1b = gcn1_b.reshape(1, D)
    g2b = gcn2_b.reshape(1, D)

    x_seq, y1, xpre1 = pl.pallas_call(
        _stage_a_body,
        grid=(GRID,),
        in_specs=[_row_spec, _w_spec, _w_spec, _w_spec, _b_spec, _deg_spec],
        out_specs=[_row_spec, _row_spec, _row_spec],
        out_shape=[_row_out, _row_out, _row_out],
    )(seq_p, W_seq, gcn1_W, lin1_W, g1b, deg_parts)

    parts1 = _agg_call(src_t, dst_t, y1, zeros_nd)

    y2, xpre2 = pl.pallas_call(
        _stage_b_body,
        grid=(GRID,),
        in_specs=[_parts_spec, _row_spec, _row_spec, _deg_spec,
                  _b_spec, _b_spec, _w_spec, _w_spec, _b_spec],
        out_specs=[_row_spec, _row_spec],
        out_shape=[_row_out, _row_out],
    )(parts1, y1, xpre1, deg_parts, ln1_g.reshape(1, D), ln1_b.reshape(1, D),
      gcn2_W, lin2_W, g2b)

    parts2 = _agg_call(src_t, dst_t, y2, zeros_nd)

    out = pl.pallas_call(
        _stage_c_body,
        grid=(GRID,),
        in_specs=[_parts_spec, _row_spec, _row_spec, _deg_spec,
                  _b_spec, _b_spec],
        out_specs=_row_spec,
        out_shape=_row_out,
    )(parts2, y2, xpre2, deg_parts, ln2_g.reshape(1, D), ln2_b.reshape(1, D))

    return (x_seq[:N], out[:N])


# speedup vs baseline: 24.1781x; 1.0132x over previous
import functools

import jax
import jax.numpy as jnp
from jax import lax
from jax.experimental import pallas as pl
from jax.experimental.pallas import tpu as pltpu
from jax.experimental.pallas import tpu_sc as plsc

N = 10000
E = 320000
D = 128

NC = 2
NS = 16
NW = NC * NS
CHUNK = 128
IB = 16
K = -(-E // (NW * CHUNK * IB)) * IB
NB = K // IB
E_PAD = NW * CHUNK * K
N_PAD = ((N + 1) + NS * CHUNK - 1) // (NS * CHUNK) * (NS * CHUNK)
RPT = N_PAD // NS

BR = 512
GRID = N_PAD // BR

_mesh = plsc.VectorSubcoreMesh(
    core_axis_name="c", subcore_axis_name="s", num_cores=NC, num_subcores=NS)


@functools.partial(
    pl.kernel,
    out_type=jax.ShapeDtypeStruct((NW, N_PAD), jnp.float32),
    mesh=_mesh,
    scratch_types=[
        pltpu.VMEM((IB, CHUNK), jnp.int32),
        pltpu.VMEM((N_PAD,), jnp.float32),
    ],
    compiler_params=pltpu.CompilerParams(needs_layout_passes=False),
)
def _deg_call(dst_hbm, out_hbm, idxb_v, hist_v):
    c = lax.axis_index("c")
    s = lax.axis_index("s")
    w = s * NC + c

    zeros16 = jnp.zeros((16,), jnp.float32)

    def zchunk(i, carry):
        hist_v[pl.ds(i * 16, 16)] = zeros16
        return carry

    lax.fori_loop(0, N_PAD // 16, zchunk, 0)

    ones16 = jnp.ones((16,), jnp.float32)

    def block(bi, carry):
        pltpu.sync_copy(dst_hbm.at[w, pl.ds(bi * IB, IB)], idxb_v)

        def chunk(j, inner):
            for q in range(CHUNK // 16):
                iv = idxb_v[j, pl.ds(q * 16, 16)]
                plsc.addupdate_scatter(hist_v, [iv], ones16)
            return inner

        lax.fori_loop(0, IB, chunk, 0)
        return carry

    lax.fori_loop(0, NB, block, 0)
    pltpu.sync_copy(hist_v, out_hbm.at[w])


@functools.partial(
    pl.kernel,
    out_type=jax.ShapeDtypeStruct((NC, N_PAD, D), jnp.float32),
    mesh=_mesh,
    scratch_types=[
        pltpu.VMEM((IB, CHUNK), jnp.int32),
        pltpu.VMEM((IB, CHUNK), jnp.int32),
        pltpu.VMEM((CHUNK, D), jnp.float32),
        pltpu.VMEM((CHUNK, D), jnp.float32),
        pltpu.VMEM_SHARED((N_PAD, D), jnp.float32),
        pltpu.SemaphoreType.DMA,
        pltpu.SemaphoreType.DMA,
        pltpu.SemaphoreType.DMA,
        pltpu.SemaphoreType.DMA,
    ],
)
def _agg_call(src_hbm, dst_hbm, y_hbm, zeros_hbm, out_hbm,
              src_v, dst_v, buf0, buf1, acc_s, gsem0, gsem1, ssem0, ssem1):
    c = lax.axis_index("c")
    s = lax.axis_index("s")
    w = s * NC + c
    r0 = s * RPT
    pltpu.sync_copy(zeros_hbm.at[pl.ds(r0, RPT)], acc_s.at[pl.ds(r0, RPT)])
    plsc.subcore_barrier()

    def block(bi, carry):
        pltpu.sync_copy(src_hbm.at[w, pl.ds(bi * IB, IB)], src_v)
        pltpu.sync_copy(dst_hbm.at[w, pl.ds(bi * IB, IB)], dst_v)

        pltpu.async_copy(y_hbm.at[src_v.at[0]], buf0, gsem0)
        pltpu.async_copy(y_hbm.at[src_v.at[1]], buf1, gsem1)

        def body(i, inner):
            j = i * 2
            pltpu.make_async_copy(y_hbm.at[src_v.at[j]], buf0, gsem0).wait()
            sc0 = pltpu.async_copy(
                buf0, acc_s.at[dst_v.at[j]], ssem0, add=True)
            pltpu.make_async_copy(y_hbm.at[src_v.at[j + 1]], buf1,
                                  gsem1).wait()
            sc1 = pltpu.async_copy(
                buf1, acc_s.at[dst_v.at[j + 1]], ssem1, add=True)
            sc0.wait()
            pltpu.async_copy(y_hbm.at[src_v.at[j + 2]], buf0, gsem0)
            sc1.wait()
            pltpu.async_copy(y_hbm.at[src_v.at[j + 3]], buf1, gsem1)
            return inner

        lax.fori_loop(0, IB // 2 - 1, body, 0)

        j = IB - 2
        pltpu.make_async_copy(y_hbm.at[src_v.at[j]], buf0, gsem0).wait()
        sc0 = pltpu.async_copy(buf0, acc_s.at[dst_v.at[j]], ssem0, add=True)
        pltpu.make_async_copy(y_hbm.at[src_v.at[j + 1]], buf1, gsem1).wait()
        sc1 = pltpu.async_copy(
            buf1, acc_s.at[dst_v.at[j + 1]], ssem1, add=True)
        sc0.wait()
        sc1.wait()
        return carry

    lax.fori_loop(0, NB, block, 0)
    plsc.subcore_barrier()
    pltpu.sync_copy(acc_s.at[pl.ds(r0, RPT)], out_hbm.at[c, pl.ds(r0, RPT)])


def _col_dis(degp):
    deg = jnp.sum(degp, axis=0)[:, None] + 1.0
    return lax.rsqrt(deg)


def _stage_a_body(seq_ref, wseq_ref, g1w_ref, l1w_ref, g1b_ref, degp_ref,
                  xseq_ref, y1_ref, xpre_ref):
    xs = jnp.maximum(
        jnp.dot(seq_ref[...], wseq_ref[...], preferred_element_type=jnp.float32),
        0.0)
    xseq_ref[...] = xs
    dis = _col_dis(degp_ref[...])
    y1_ref[...] = dis * jnp.dot(xs, g1w_ref[...],
                                preferred_element_type=jnp.float32)
    xpre_ref[...] = (jnp.dot(xs, l1w_ref[...],
                             preferred_element_type=jnp.float32)
                     + g1b_ref[...] + 1e-6)


def _layer_norm_block(x, g, b):
    mu = jnp.mean(x, axis=-1, keepdims=True)
    xc = x - mu
    var = jnp.mean(xc * xc, axis=-1, keepdims=True)
    return xc * lax.rsqrt(var + 1e-5) * g + b


def _stage_b_body(parts_ref, y1_ref, xpre_ref, degp_ref, g_ref, b_ref,
                  g2w_ref, l2w_ref, g2b_ref, y2_ref, xpre2_ref):
    dis = _col_dis(degp_ref[...])
    acc = parts_ref[0] + parts_ref[1] + y1_ref[...]
    s = dis * acc + xpre_ref[...]
    x = jnp.maximum(_layer_norm_block(s, g_ref[...], b_ref[...]), 0.0)
    y2_ref[...] = dis * jnp.dot(x, g2w_ref[...],
                                preferred_element_type=jnp.float32)
    xpre2_ref[...] = (jnp.dot(x, l2w_ref[...],
                              preferred_element_type=jnp.float32)
                      + g2b_ref[...] + 1e-6)


def _stage_c_body(parts_ref, y2_ref, xpre2_ref, degp_ref, g_ref, b_ref,
                  out_ref):
    dis = _col_dis(degp_ref[...])
    acc = parts_ref[0] + parts_ref[1] + y2_ref[...]
    s = dis * acc + xpre2_ref[...]
    out_ref[...] = _layer_norm_block(s, g_ref[...], b_ref[...])


_row_spec = pl.BlockSpec((BR, D), lambda i: (i, 0))
_w_spec = pl.BlockSpec((D, D), lambda i: (0, 0))
_b_spec = pl.BlockSpec((1, D), lambda i: (0, 0))
_deg_spec = pl.BlockSpec((NW, BR), lambda i: (0, i))
_parts_spec = pl.BlockSpec((2, BR, D), lambda i: (0, i, 0))
_row_out = jax.ShapeDtypeStruct((N_PAD, D), jnp.float32)


def kernel(mat, seq, W_seq, gcn1_W, gcn1_b, lin1_W, ln1_g, ln1_b,
           gcn2_W, gcn2_b, lin2_W, ln2_g, ln2_b):
    src = mat[0]
    dst = mat[1]
    pad = E_PAD - E
    ept = E // NW
    ppt = K * CHUNK - ept
    pad_idx = (N + jnp.arange(NW * ppt, dtype=jnp.int32) % (N_PAD - N)
               ).reshape(NW, ppt)
    src_t = jnp.concatenate(
        [src.reshape(NW, ept), pad_idx], axis=1).reshape(NW, K, CHUNK)
    dst_t = jnp.concatenate(
        [dst.reshape(NW, ept), pad_idx], axis=1).reshape(NW, K, CHUNK)

    zeros_nd = jnp.zeros((N_PAD, D), jnp.float32)
    seq_p = jnp.concatenate(
        [seq, jnp.zeros((N_PAD - N, D), jnp.float32)], axis=0)

    deg_parts = _deg_call(dst_t)

    g1b = gcn1_b.reshape(1, D)
    g2b = gcn2_b.reshape(1, D)

    x_seq, y1, xpre1 = pl.pallas_call(
        _stage_a_body,
        grid=(GRID,),
        in_specs=[_row_spec, _w_spec, _w_spec, _w_spec, _b_spec, _deg_spec],
        out_specs=[_row_spec, _row_spec, _row_spec],
        out_shape=[_row_out, _row_out, _row_out],
    )(seq_p, W_seq, gcn1_W, lin1_W, g1b, deg_parts)

    parts1 = _agg_call(src_t, dst_t, y1, zeros_nd)

    y2, xpre2 = pl.pallas_call(
        _stage_b_body,
        grid=(GRID,),
        in_specs=[_parts_spec, _row_spec, _row_spec, _deg_spec,
                  _b_spec, _b_spec, _w_spec, _w_spec, _b_spec],
        out_specs=[_row_spec, _row_spec],
        out_shape=[_row_out, _row_out],
    )(parts1, y1, xpre1, deg_parts, ln1_g.reshape(1, D), ln1_b.reshape(1, D),
      gcn2_W, lin2_W, g2b)

    parts2 = _agg_call(src_t, dst_t, y2, zeros_nd)

    out = pl.pallas_call(
        _stage_c_body,
        grid=(GRID,),
        in_specs=[_parts_spec, _row_spec, _row_spec, _deg_spec,
                  _b_spec, _b_spec],
        out_specs=_row_spec,
        out_shape=_row_out,
    )(parts2, y2, xpre2, deg_parts, ln2_g.reshape(1, D), ln2_b.reshape(1, D))

    return (x_seq[:N], out[:N])


# speedup vs baseline: 25.2688x; 1.0451x over previous
import functools

import jax
import jax.numpy as jnp
from jax import lax
from jax.experimental import pallas as pl
from jax.experimental.pallas import tpu as pltpu
from jax.experimental.pallas import tpu_sc as plsc

N = 10000
E = 320000
D = 128

NC = 2
NS = 16
NW = NC * NS
CHUNK = 128
IB = 16
K = -(-E // (NW * CHUNK * IB)) * IB
NB = K // IB
E_PAD = NW * CHUNK * K
N_PAD = ((N + 1) + NS * CHUNK - 1) // (NS * CHUNK) * (NS * CHUNK)
RPT = N_PAD // NS

BR = 1024
GRID = N_PAD // BR

_mesh = plsc.VectorSubcoreMesh(
    core_axis_name="c", subcore_axis_name="s", num_cores=NC, num_subcores=NS)


@functools.partial(
    pl.kernel,
    out_type=jax.ShapeDtypeStruct((NW, N_PAD), jnp.float32),
    mesh=_mesh,
    scratch_types=[
        pltpu.VMEM((IB, CHUNK), jnp.int32),
        pltpu.VMEM((N_PAD,), jnp.float32),
    ],
    compiler_params=pltpu.CompilerParams(needs_layout_passes=False),
)
def _deg_call(dst_hbm, out_hbm, idxb_v, hist_v):
    c = lax.axis_index("c")
    s = lax.axis_index("s")
    w = s * NC + c

    zeros16 = jnp.zeros((16,), jnp.float32)

    def zchunk(i, carry):
        hist_v[pl.ds(i * 16, 16)] = zeros16
        return carry

    lax.fori_loop(0, N_PAD // 16, zchunk, 0)

    ones16 = jnp.ones((16,), jnp.float32)

    def block(bi, carry):
        pltpu.sync_copy(dst_hbm.at[w, pl.ds(bi * IB, IB)], idxb_v)

        def chunk(j, inner):
            for q in range(CHUNK // 16):
                iv = idxb_v[j, pl.ds(q * 16, 16)]
                plsc.addupdate_scatter(hist_v, [iv], ones16)
            return inner

        lax.fori_loop(0, IB, chunk, 0)
        return carry

    lax.fori_loop(0, NB, block, 0)
    pltpu.sync_copy(hist_v, out_hbm.at[w])


@functools.partial(
    pl.kernel,
    out_type=jax.ShapeDtypeStruct((NC, N_PAD, D), jnp.float32),
    mesh=_mesh,
    scratch_types=[
        pltpu.VMEM((IB, CHUNK), jnp.int32),
        pltpu.VMEM((IB, CHUNK), jnp.int32),
        pltpu.VMEM((CHUNK, D), jnp.float32),
        pltpu.VMEM((CHUNK, D), jnp.float32),
        pltpu.VMEM_SHARED((N_PAD, D), jnp.float32),
        pltpu.SemaphoreType.DMA,
        pltpu.SemaphoreType.DMA,
        pltpu.SemaphoreType.DMA,
        pltpu.SemaphoreType.DMA,
    ],
)
def _agg_call(src_hbm, dst_hbm, y_hbm, zeros_hbm, out_hbm,
              src_v, dst_v, buf0, buf1, acc_s, gsem0, gsem1, ssem0, ssem1):
    c = lax.axis_index("c")
    s = lax.axis_index("s")
    w = s * NC + c
    r0 = s * RPT
    pltpu.sync_copy(zeros_hbm.at[pl.ds(r0, RPT)], acc_s.at[pl.ds(r0, RPT)])
    plsc.subcore_barrier()

    def block(bi, carry):
        pltpu.sync_copy(src_hbm.at[w, pl.ds(bi * IB, IB)], src_v)
        pltpu.sync_copy(dst_hbm.at[w, pl.ds(bi * IB, IB)], dst_v)

        pltpu.async_copy(y_hbm.at[src_v.at[0]], buf0, gsem0)
        pltpu.async_copy(y_hbm.at[src_v.at[1]], buf1, gsem1)

        def body(i, inner):
            j = i * 2
            pltpu.make_async_copy(y_hbm.at[src_v.at[j]], buf0, gsem0).wait()
            sc0 = pltpu.async_copy(
                buf0, acc_s.at[dst_v.at[j]], ssem0, add=True)
            pltpu.make_async_copy(y_hbm.at[src_v.at[j + 1]], buf1,
                                  gsem1).wait()
            sc1 = pltpu.async_copy(
                buf1, acc_s.at[dst_v.at[j + 1]], ssem1, add=True)
            sc0.wait()
            pltpu.async_copy(y_hbm.at[src_v.at[j + 2]], buf0, gsem0)
            sc1.wait()
            pltpu.async_copy(y_hbm.at[src_v.at[j + 3]], buf1, gsem1)
            return inner

        lax.fori_loop(0, IB // 2 - 1, body, 0)

        j = IB - 2
        pltpu.make_async_copy(y_hbm.at[src_v.at[j]], buf0, gsem0).wait()
        sc0 = pltpu.async_copy(buf0, acc_s.at[dst_v.at[j]], ssem0, add=True)
        pltpu.make_async_copy(y_hbm.at[src_v.at[j + 1]], buf1, gsem1).wait()
        sc1 = pltpu.async_copy(
            buf1, acc_s.at[dst_v.at[j + 1]], ssem1, add=True)
        sc0.wait()
        sc1.wait()
        return carry

    lax.fori_loop(0, NB, block, 0)
    plsc.subcore_barrier()
    pltpu.sync_copy(acc_s.at[pl.ds(r0, RPT)], out_hbm.at[c, pl.ds(r0, RPT)])


def _col_dis(degp):
    deg = jnp.sum(degp, axis=0)[:, None] + 1.0
    return lax.rsqrt(deg)


def _stage_a_body(seq_ref, wseq_ref, g1w_ref, l1w_ref, g1b_ref, degp_ref,
                  xseq_ref, y1_ref, xpre_ref):
    xs = jnp.maximum(
        jnp.dot(seq_ref[...], wseq_ref[...], preferred_element_type=jnp.float32),
        0.0)
    xseq_ref[...] = xs
    dis = _col_dis(degp_ref[...])
    y1_ref[...] = dis * jnp.dot(xs, g1w_ref[...],
                                preferred_element_type=jnp.float32)
    xpre_ref[...] = (jnp.dot(xs, l1w_ref[...],
                             preferred_element_type=jnp.float32)
                     + g1b_ref[...] + 1e-6)


def _layer_norm_block(x, g, b):
    mu = jnp.mean(x, axis=-1, keepdims=True)
    xc = x - mu
    var = jnp.mean(xc * xc, axis=-1, keepdims=True)
    return xc * lax.rsqrt(var + 1e-5) * g + b


def _stage_b_body(parts_ref, y1_ref, xpre_ref, degp_ref, g_ref, b_ref,
                  g2w_ref, l2w_ref, g2b_ref, y2_ref, xpre2_ref):
    dis = _col_dis(degp_ref[...])
    acc = parts_ref[0] + parts_ref[1] + y1_ref[...]
    s = dis * acc + xpre_ref[...]
    x = jnp.maximum(_layer_norm_block(s, g_ref[...], b_ref[...]), 0.0)
    y2_ref[...] = dis * jnp.dot(x, g2w_ref[...],
                                preferred_element_type=jnp.float32)
    xpre2_ref[...] = (jnp.dot(x, l2w_ref[...],
                              preferred_element_type=jnp.float32)
                      + g2b_ref[...] + 1e-6)


def _stage_c_body(parts_ref, y2_ref, xpre2_ref, degp_ref, g_ref, b_ref,
                  out_ref):
    dis = _col_dis(degp_ref[...])
    acc = parts_ref[0] + parts_ref[1] + y2_ref[...]
    s = dis * acc + xpre2_ref[...]
    out_ref[...] = _layer_norm_block(s, g_ref[...], b_ref[...])


_row_spec = pl.BlockSpec((BR, D), lambda i: (i, 0))
_w_spec = pl.BlockSpec((D, D), lambda i: (0, 0))
_b_spec = pl.BlockSpec((1, D), lambda i: (0, 0))
_deg_spec = pl.BlockSpec((NW, BR), lambda i: (0, i))
_parts_spec = pl.BlockSpec((2, BR, D), lambda i: (0, i, 0))
_row_out = jax.ShapeDtypeStruct((N_PAD, D), jnp.float32)


def kernel(mat, seq, W_seq, gcn1_W, gcn1_b, lin1_W, ln1_g, ln1_b,
           gcn2_W, gcn2_b, lin2_W, ln2_g, ln2_b):
    src = mat[0]
    dst = mat[1]
    pad = E_PAD - E
    ept = E // NW
    ppt = K * CHUNK - ept
    pad_idx = (N + jnp.arange(NW * ppt, dtype=jnp.int32) % (N_PAD - N)
               ).reshape(NW, ppt)
    src_t = jnp.concatenate(
        [src.reshape(NW, ept), pad_idx], axis=1).reshape(NW, K, CHUNK)
    dst_t = jnp.concatenate(
        [dst.reshape(NW, ept), pad_idx], axis=1).reshape(NW, K, CHUNK)

    zeros_nd = jnp.zeros((N_PAD, D), jnp.float32)
    seq_p = jnp.concatenate(
        [seq, jnp.zeros((N_PAD - N, D), jnp.float32)], axis=0)

    deg_parts = _deg_call(dst_t)

    g1b = gcn1_b.reshape(1, D)
    g2b = gcn2_b.reshape(1, D)

    x_seq, y1, xpre1 = pl.pallas_call(
        _stage_a_body,
        grid=(GRID,),
        in_specs=[_row_spec, _w_spec, _w_spec, _w_spec, _b_spec, _deg_spec],
        out_specs=[_row_spec, _row_spec, _row_spec],
        out_shape=[_row_out, _row_out, _row_out],
    )(seq_p, W_seq, gcn1_W, lin1_W, g1b, deg_parts)

    parts1 = _agg_call(src_t, dst_t, y1, zeros_nd)

    y2, xpre2 = pl.pallas_call(
        _stage_b_body,
        grid=(GRID,),
        in_specs=[_parts_spec, _row_spec, _row_spec, _deg_spec,
                  _b_spec, _b_spec, _w_spec, _w_spec, _b_spec],
        out_specs=[_row_spec, _row_spec],
        out_shape=[_row_out, _row_out],
    )(parts1, y1, xpre1, deg_parts, ln1_g.reshape(1, D), ln1_b.reshape(1, D),
      gcn2_W, lin2_W, g2b)

    parts2 = _agg_call(src_t, dst_t, y2, zeros_nd)

    out = pl.pallas_call(
        _stage_c_body,
        grid=(GRID,),
        in_specs=[_parts_spec, _row_spec, _row_spec, _deg_spec,
                  _b_spec, _b_spec],
        out_specs=_row_spec,
        out_shape=_row_out,
    )(parts2, y2, xpre2, deg_parts, ln2_g.reshape(1, D), ln2_b.reshape(1, D))

    return (x_seq[:N], out[:N])


# speedup vs baseline: 25.9218x; 1.0258x over previous
import functools

import jax
import jax.numpy as jnp
from jax import lax
from jax.experimental import pallas as pl
from jax.experimental.pallas import tpu as pltpu
from jax.experimental.pallas import tpu_sc as plsc

N = 10000
E = 320000
D = 128

NC = 2
NS = 16
NW = NC * NS
CHUNK = 128
IB = 16
K = -(-E // (NW * CHUNK * IB)) * IB
NB = K // IB
E_PAD = NW * CHUNK * K
N_PAD = ((N + 1) + NS * CHUNK - 1) // (NS * CHUNK) * (NS * CHUNK)
RPT = N_PAD // NS

BR = 2048
GRID = N_PAD // BR

_mesh = plsc.VectorSubcoreMesh(
    core_axis_name="c", subcore_axis_name="s", num_cores=NC, num_subcores=NS)


@functools.partial(
    pl.kernel,
    out_type=jax.ShapeDtypeStruct((NW, N_PAD), jnp.float32),
    mesh=_mesh,
    scratch_types=[
        pltpu.VMEM((IB, CHUNK), jnp.int32),
        pltpu.VMEM((N_PAD,), jnp.float32),
    ],
    compiler_params=pltpu.CompilerParams(needs_layout_passes=False),
)
def _deg_call(dst_hbm, out_hbm, idxb_v, hist_v):
    c = lax.axis_index("c")
    s = lax.axis_index("s")
    w = s * NC + c

    zeros16 = jnp.zeros((16,), jnp.float32)

    def zchunk(i, carry):
        hist_v[pl.ds(i * 16, 16)] = zeros16
        return carry

    lax.fori_loop(0, N_PAD // 16, zchunk, 0)

    ones16 = jnp.ones((16,), jnp.float32)

    def block(bi, carry):
        pltpu.sync_copy(dst_hbm.at[w, pl.ds(bi * IB, IB)], idxb_v)

        def chunk(j, inner):
            for q in range(CHUNK // 16):
                iv = idxb_v[j, pl.ds(q * 16, 16)]
                plsc.addupdate_scatter(hist_v, [iv], ones16)
            return inner

        lax.fori_loop(0, IB, chunk, 0)
        return carry

    lax.fori_loop(0, NB, block, 0)
    pltpu.sync_copy(hist_v, out_hbm.at[w])


@functools.partial(
    pl.kernel,
    out_type=jax.ShapeDtypeStruct((NC, N_PAD, D), jnp.float32),
    mesh=_mesh,
    scratch_types=[
        pltpu.VMEM((IB, CHUNK), jnp.int32),
        pltpu.VMEM((IB, CHUNK), jnp.int32),
        pltpu.VMEM((CHUNK, D), jnp.float32),
        pltpu.VMEM((CHUNK, D), jnp.float32),
        pltpu.VMEM_SHARED((N_PAD, D), jnp.float32),
        pltpu.SemaphoreType.DMA,
        pltpu.SemaphoreType.DMA,
        pltpu.SemaphoreType.DMA,
        pltpu.SemaphoreType.DMA,
    ],
)
def _agg_call(src_hbm, dst_hbm, y_hbm, zeros_hbm, out_hbm,
              src_v, dst_v, buf0, buf1, acc_s, gsem0, gsem1, ssem0, ssem1):
    c = lax.axis_index("c")
    s = lax.axis_index("s")
    w = s * NC + c
    r0 = s * RPT
    pltpu.sync_copy(zeros_hbm.at[pl.ds(r0, RPT)], acc_s.at[pl.ds(r0, RPT)])
    plsc.subcore_barrier()

    def block(bi, carry):
        pltpu.sync_copy(src_hbm.at[w, pl.ds(bi * IB, IB)], src_v)
        pltpu.sync_copy(dst_hbm.at[w, pl.ds(bi * IB, IB)], dst_v)

        pltpu.async_copy(y_hbm.at[src_v.at[0]], buf0, gsem0)
        pltpu.async_copy(y_hbm.at[src_v.at[1]], buf1, gsem1)

        def body(i, inner):
            j = i * 2
            pltpu.make_async_copy(y_hbm.at[src_v.at[j]], buf0, gsem0).wait()
            sc0 = pltpu.async_copy(
                buf0, acc_s.at[dst_v.at[j]], ssem0, add=True)
            pltpu.make_async_copy(y_hbm.at[src_v.at[j + 1]], buf1,
                                  gsem1).wait()
            sc1 = pltpu.async_copy(
                buf1, acc_s.at[dst_v.at[j + 1]], ssem1, add=True)
            sc0.wait()
            pltpu.async_copy(y_hbm.at[src_v.at[j + 2]], buf0, gsem0)
            sc1.wait()
            pltpu.async_copy(y_hbm.at[src_v.at[j + 3]], buf1, gsem1)
            return inner

        lax.fori_loop(0, IB // 2 - 1, body, 0)

        j = IB - 2
        pltpu.make_async_copy(y_hbm.at[src_v.at[j]], buf0, gsem0).wait()
        sc0 = pltpu.async_copy(buf0, acc_s.at[dst_v.at[j]], ssem0, add=True)
        pltpu.make_async_copy(y_hbm.at[src_v.at[j + 1]], buf1, gsem1).wait()
        sc1 = pltpu.async_copy(
            buf1, acc_s.at[dst_v.at[j + 1]], ssem1, add=True)
        sc0.wait()
        sc1.wait()
        return carry

    lax.fori_loop(0, NB, block, 0)
    plsc.subcore_barrier()
    pltpu.sync_copy(acc_s.at[pl.ds(r0, RPT)], out_hbm.at[c, pl.ds(r0, RPT)])


def _col_dis(degp):
    deg = jnp.sum(degp, axis=0)[:, None] + 1.0
    return lax.rsqrt(deg)


def _stage_a_body(seq_ref, wseq_ref, g1w_ref, l1w_ref, g1b_ref, degp_ref,
                  xseq_ref, y1_ref, xpre_ref):
    xs = jnp.maximum(
        jnp.dot(seq_ref[...], wseq_ref[...], preferred_element_type=jnp.float32),
        0.0)
    xseq_ref[...] = xs
    dis = _col_dis(degp_ref[...])
    y1_ref[...] = dis * jnp.dot(xs, g1w_ref[...],
                                preferred_element_type=jnp.float32)
    xpre_ref[...] = (jnp.dot(xs, l1w_ref[...],
                             preferred_element_type=jnp.float32)
                     + g1b_ref[...] + 1e-6)


def _layer_norm_block(x, g, b):
    mu = jnp.mean(x, axis=-1, keepdims=True)
    xc = x - mu
    var = jnp.mean(xc * xc, axis=-1, keepdims=True)
    return xc * lax.rsqrt(var + 1e-5) * g + b


def _stage_b_body(parts_ref, y1_ref, xpre_ref, degp_ref, g_ref, b_ref,
                  g2w_ref, l2w_ref, g2b_ref, y2_ref, xpre2_ref):
    dis = _col_dis(degp_ref[...])
    acc = parts_ref[0] + parts_ref[1] + y1_ref[...]
    s = dis * acc + xpre_ref[...]
    x = jnp.maximum(_layer_norm_block(s, g_ref[...], b_ref[...]), 0.0)
    y2_ref[...] = dis * jnp.dot(x, g2w_ref[...],
                                preferred_element_type=jnp.float32)
    xpre2_ref[...] = (jnp.dot(x, l2w_ref[...],
                              preferred_element_type=jnp.float32)
                      + g2b_ref[...] + 1e-6)


def _stage_c_body(parts_ref, y2_ref, xpre2_ref, degp_ref, g_ref, b_ref,
                  out_ref):
    dis = _col_dis(degp_ref[...])
    acc = parts_ref[0] + parts_ref[1] + y2_ref[...]
    s = dis * acc + xpre2_ref[...]
    out_ref[...] = _layer_norm_block(s, g_ref[...], b_ref[...])


_row_spec = pl.BlockSpec((BR, D), lambda i: (i, 0))
_w_spec = pl.BlockSpec((D, D), lambda i: (0, 0))
_b_spec = pl.BlockSpec((1, D), lambda i: (0, 0))
_deg_spec = pl.BlockSpec((NW, BR), lambda i: (0, i))
_parts_spec = pl.BlockSpec((2, BR, D), lambda i: (0, i, 0))
_row_out = jax.ShapeDtypeStruct((N_PAD, D), jnp.float32)


def kernel(mat, seq, W_seq, gcn1_W, gcn1_b, lin1_W, ln1_g, ln1_b,
           gcn2_W, gcn2_b, lin2_W, ln2_g, ln2_b):
    src = mat[0]
    dst = mat[1]
    pad = E_PAD - E
    ept = E // NW
    ppt = K * CHUNK - ept
    pad_idx = (N + jnp.arange(NW * ppt, dtype=jnp.int32) % (N_PAD - N)
               ).reshape(NW, ppt)
    src_t = jnp.concatenate(
        [src.reshape(NW, ept), pad_idx], axis=1).reshape(NW, K, CHUNK)
    dst_t = jnp.concatenate(
        [dst.reshape(NW, ept), pad_idx], axis=1).reshape(NW, K, CHUNK)

    zeros_nd = jnp.zeros((N_PAD, D), jnp.float32)
    seq_p = jnp.concatenate(
        [seq, jnp.zeros((N_PAD - N, D), jnp.float32)], axis=0)

    deg_parts = _deg_call(dst_t)

    g1b = gcn1_b.reshape(1, D)
    g2b = gcn2_b.reshape(1, D)

    x_seq, y1, xpre1 = pl.pallas_call(
        _stage_a_body,
        grid=(GRID,),
        in_specs=[_row_spec, _w_spec, _w_spec, _w_spec, _b_spec, _deg_spec],
        out_specs=[_row_spec, _row_spec, _row_spec],
        out_shape=[_row_out, _row_out, _row_out],
    )(seq_p, W_seq, gcn1_W, lin1_W, g1b, deg_parts)

    parts1 = _agg_call(src_t, dst_t, y1, zeros_nd)

    y2, xpre2 = pl.pallas_call(
        _stage_b_body,
        grid=(GRID,),
        in_specs=[_parts_spec, _row_spec, _row_spec, _deg_spec,
                  _b_spec, _b_spec, _w_spec, _w_spec, _b_spec],
        out_specs=[_row_spec, _row_spec],
        out_shape=[_row_out, _row_out],
    )(parts1, y1, xpre1, deg_parts, ln1_g.reshape(1, D), ln1_b.reshape(1, D),
      gcn2_W, lin2_W, g2b)

    parts2 = _agg_call(src_t, dst_t, y2, zeros_nd)

    out = pl.pallas_call(
        _stage_c_body,
        grid=(GRID,),
        in_specs=[_parts_spec, _row_spec, _row_spec, _deg_spec,
                  _b_spec, _b_spec],
        out_specs=_row_spec,
        out_shape=_row_out,
    )(parts2, y2, xpre2, deg_parts, ln2_g.reshape(1, D), ln2_b.reshape(1, D))

    return (x_seq[:N], out[:N])


# speedup vs baseline: 26.0507x; 1.0050x over previous
import functools

import jax
import jax.numpy as jnp
from jax import lax
from jax.experimental import pallas as pl
from jax.experimental.pallas import tpu as pltpu
from jax.experimental.pallas import tpu_sc as plsc

N = 10000
E = 320000
D = 128

NC = 2
NS = 16
NW = NC * NS
CHUNK = 128
IB = 16
K = -(-E // (NW * CHUNK * IB)) * IB
NB = K // IB
E_PAD = NW * CHUNK * K
N_PAD = ((N + 1) + NS * CHUNK - 1) // (NS * CHUNK) * (NS * CHUNK)
RPT = N_PAD // NS

BR = 2560
GRID = N_PAD // BR

_mesh = plsc.VectorSubcoreMesh(
    core_axis_name="c", subcore_axis_name="s", num_cores=NC, num_subcores=NS)


@functools.partial(
    pl.kernel,
    out_type=jax.ShapeDtypeStruct((NW, N_PAD), jnp.float32),
    mesh=_mesh,
    scratch_types=[
        pltpu.VMEM((IB, CHUNK), jnp.int32),
        pltpu.VMEM((N_PAD,), jnp.float32),
    ],
    compiler_params=pltpu.CompilerParams(needs_layout_passes=False),
)
def _deg_call(dst_hbm, out_hbm, idxb_v, hist_v):
    c = lax.axis_index("c")
    s = lax.axis_index("s")
    w = s * NC + c

    zeros16 = jnp.zeros((16,), jnp.float32)

    def zchunk(i, carry):
        hist_v[pl.ds(i * 16, 16)] = zeros16
        return carry

    lax.fori_loop(0, N_PAD // 16, zchunk, 0)

    ones16 = jnp.ones((16,), jnp.float32)

    def block(bi, carry):
        pltpu.sync_copy(dst_hbm.at[w, pl.ds(bi * IB, IB)], idxb_v)

        def chunk(j, inner):
            for q in range(CHUNK // 16):
                iv = idxb_v[j, pl.ds(q * 16, 16)]
                plsc.addupdate_scatter(hist_v, [iv], ones16)
            return inner

        lax.fori_loop(0, IB, chunk, 0)
        return carry

    lax.fori_loop(0, NB, block, 0)
    pltpu.sync_copy(hist_v, out_hbm.at[w])


@functools.partial(
    pl.kernel,
    out_type=jax.ShapeDtypeStruct((NC, N_PAD, D), jnp.float32),
    mesh=_mesh,
    scratch_types=[
        pltpu.VMEM((IB, CHUNK), jnp.int32),
        pltpu.VMEM((IB, CHUNK), jnp.int32),
        pltpu.VMEM((CHUNK, D), jnp.float32),
        pltpu.VMEM((CHUNK, D), jnp.float32),
        pltpu.VMEM_SHARED((N_PAD, D), jnp.float32),
        pltpu.SemaphoreType.DMA,
        pltpu.SemaphoreType.DMA,
        pltpu.SemaphoreType.DMA,
        pltpu.SemaphoreType.DMA,
    ],
)
def _agg_call(src_hbm, dst_hbm, y_hbm, zeros_hbm, out_hbm,
              src_v, dst_v, buf0, buf1, acc_s, gsem0, gsem1, ssem0, ssem1):
    c = lax.axis_index("c")
    s = lax.axis_index("s")
    w = s * NC + c
    r0 = s * RPT
    pltpu.sync_copy(zeros_hbm.at[pl.ds(r0, RPT)], acc_s.at[pl.ds(r0, RPT)])
    plsc.subcore_barrier()

    def block(bi, carry):
        pltpu.sync_copy(src_hbm.at[w, pl.ds(bi * IB, IB)], src_v)
        pltpu.sync_copy(dst_hbm.at[w, pl.ds(bi * IB, IB)], dst_v)

        pltpu.async_copy(y_hbm.at[src_v.at[0]], buf0, gsem0)
        pltpu.async_copy(y_hbm.at[src_v.at[1]], buf1, gsem1)

        def body(i, inner):
            j = i * 2
            pltpu.make_async_copy(y_hbm.at[src_v.at[j]], buf0, gsem0).wait()
            sc0 = pltpu.async_copy(
                buf0, acc_s.at[dst_v.at[j]], ssem0, add=True)
            pltpu.make_async_copy(y_hbm.at[src_v.at[j + 1]], buf1,
                                  gsem1).wait()
            sc1 = pltpu.async_copy(
                buf1, acc_s.at[dst_v.at[j + 1]], ssem1, add=True)
            sc0.wait()
            pltpu.async_copy(y_hbm.at[src_v.at[j + 2]], buf0, gsem0)
            sc1.wait()
            pltpu.async_copy(y_hbm.at[src_v.at[j + 3]], buf1, gsem1)
            return inner

        lax.fori_loop(0, IB // 2 - 1, body, 0)

        j = IB - 2
        pltpu.make_async_copy(y_hbm.at[src_v.at[j]], buf0, gsem0).wait()
        sc0 = pltpu.async_copy(buf0, acc_s.at[dst_v.at[j]], ssem0, add=True)
        pltpu.make_async_copy(y_hbm.at[src_v.at[j + 1]], buf1, gsem1).wait()
        sc1 = pltpu.async_copy(
            buf1, acc_s.at[dst_v.at[j + 1]], ssem1, add=True)
        sc0.wait()
        sc1.wait()
        return carry

    lax.fori_loop(0, NB, block, 0)
    plsc.subcore_barrier()
    pltpu.sync_copy(acc_s.at[pl.ds(r0, RPT)], out_hbm.at[c, pl.ds(r0, RPT)])


def _col_dis(degp):
    deg = jnp.sum(degp, axis=0)[:, None] + 1.0
    return lax.rsqrt(deg)


def _stage_a_body(seq_ref, wseq_ref, g1w_ref, l1w_ref, g1b_ref, degp_ref,
                  xseq_ref, y1_ref, xpre_ref):
    xs = jnp.maximum(
        jnp.dot(seq_ref[...], wseq_ref[...], preferred_element_type=jnp.float32),
        0.0)
    xseq_ref[...] = xs
    dis = _col_dis(degp_ref[...])
    y1_ref[...] = dis * jnp.dot(xs, g1w_ref[...],
                                preferred_element_type=jnp.float32)
    xpre_ref[...] = (jnp.dot(xs, l1w_ref[...],
                             preferred_element_type=jnp.float32)
                     + g1b_ref[...] + 1e-6)


def _layer_norm_block(x, g, b):
    mu = jnp.mean(x, axis=-1, keepdims=True)
    xc = x - mu
    var = jnp.mean(xc * xc, axis=-1, keepdims=True)
    return xc * lax.rsqrt(var + 1e-5) * g + b


def _stage_b_body(parts_ref, y1_ref, xpre_ref, degp_ref, g_ref, b_ref,
                  g2w_ref, l2w_ref, g2b_ref, y2_ref, xpre2_ref):
    dis = _col_dis(degp_ref[...])
    acc = parts_ref[0] + parts_ref[1] + y1_ref[...]
    s = dis * acc + xpre_ref[...]
    x = jnp.maximum(_layer_norm_block(s, g_ref[...], b_ref[...]), 0.0)
    y2_ref[...] = dis * jnp.dot(x, g2w_ref[...],
                                preferred_element_type=jnp.float32)
    xpre2_ref[...] = (jnp.dot(x, l2w_ref[...],
                              preferred_element_type=jnp.float32)
                      + g2b_ref[...] + 1e-6)


def _stage_c_body(parts_ref, y2_ref, xpre2_ref, degp_ref, g_ref, b_ref,
                  out_ref):
    dis = _col_dis(degp_ref[...])
    acc = parts_ref[0] + parts_ref[1] + y2_ref[...]
    s = dis * acc + xpre2_ref[...]
    out_ref[...] = _layer_norm_block(s, g_ref[...], b_ref[...])


_row_spec = pl.BlockSpec((BR, D), lambda i: (i, 0))
_w_spec = pl.BlockSpec((D, D), lambda i: (0, 0))
_b_spec = pl.BlockSpec((1, D), lambda i: (0, 0))
_deg_spec = pl.BlockSpec((NW, BR), lambda i: (0, i))
_parts_spec = pl.BlockSpec((2, BR, D), lambda i: (0, i, 0))
_row_out = jax.ShapeDtypeStruct((N_PAD, D), jnp.float32)


def kernel(mat, seq, W_seq, gcn1_W, gcn1_b, lin1_W, ln1_g, ln1_b,
           gcn2_W, gcn2_b, lin2_W, ln2_g, ln2_b):
    src = mat[0]
    dst = mat[1]
    pad = E_PAD - E
    ept = E // NW
    ppt = K * CHUNK - ept
    pad_idx = (N + jnp.arange(NW * ppt, dtype=jnp.int32) % (N_PAD - N)
               ).reshape(NW, ppt)
    src_t = jnp.concatenate(
        [src.reshape(NW, ept), pad_idx], axis=1).reshape(NW, K, CHUNK)
    dst_t = jnp.concatenate(
        [dst.reshape(NW, ept), pad_idx], axis=1).reshape(NW, K, CHUNK)

    zeros_nd = jnp.zeros((N_PAD, D), jnp.float32)
    seq_p = jnp.concatenate(
        [seq, jnp.zeros((N_PAD - N, D), jnp.float32)], axis=0)

    deg_parts = _deg_call(dst_t)

    g1b = gcn1_b.reshape(1, D)
    g2b = gcn2_b.reshape(1, D)

    x_seq, y1, xpre1 = pl.pallas_call(
        _stage_a_body,
        grid=(GRID,),
        in_specs=[_row_spec, _w_spec, _w_spec, _w_spec, _b_spec, _deg_spec],
        out_specs=[_row_spec, _row_spec, _row_spec],
        out_shape=[_row_out, _row_out, _row_out],
    )(seq_p, W_seq, gcn1_W, lin1_W, g1b, deg_parts)

    parts1 = _agg_call(src_t, dst_t, y1, zeros_nd)

    y2, xpre2 = pl.pallas_call(
        _stage_b_body,
        grid=(GRID,),
        in_specs=[_parts_spec, _row_spec, _row_spec, _deg_spec,
                  _b_spec, _b_spec, _w_spec, _w_spec, _b_spec],
        out_specs=[_row_spec, _row_spec],
        out_shape=[_row_out, _row_out],
    )(parts1, y1, xpre1, deg_parts, ln1_g.reshape(1, D), ln1_b.reshape(1, D),
      gcn2_W, lin2_W, g2b)

    parts2 = _agg_call(src_t, dst_t, y2, zeros_nd)

    out = pl.pallas_call(
        _stage_c_body,
        grid=(GRID,),
        in_specs=[_parts_spec, _row_spec, _row_spec, _deg_spec,
                  _b_spec, _b_spec],
        out_specs=_row_spec,
        out_shape=_row_out,
    )(parts2, y2, xpre2, deg_parts, ln2_g.reshape(1, D), ln2_b.reshape(1, D))

    return (x_seq[:N], out[:N])
